# Initial kernel scaffold; baseline (speedup 1.0000x reference)
import jax
import jax.numpy as jnp
from jax.experimental import pallas as pl


def kernel(child_feats, child_exists, edge_type_onehot, edge_feats, edge_indices, W_child, b_child, W_ne, b_ne, W_parent, b_parent):
    raise NotImplementedError("write your pallas kernel here")


# speedup vs baseline: 3.0578x; 3.0578x over previous
import functools

import jax
import jax.numpy as jnp
from jax import lax
from jax.experimental import pallas as pl
from jax.experimental.pallas import tpu as pltpu
from jax.experimental.pallas import tpu_sc as plsc

N = 10000
E = 320000
DF = 128
DH = 128
DEF = 20
ACC_W = 144

NC, NS = 2, 16
NW = NC * NS
EPW = E // NW
K = 40
NBLK = EPW // K
NPAD = 10240
RPT = NPAD // NS
RZC = 128


def _tc_pre_body(child_ref, exists_ref, wct_ref, bc_ref, w1t_ref, w2t_ref,
                 a_ref, b_ref, psum_ref, esum_ref):
    i = pl.program_id(0)
    x = child_ref[...]
    cf = jnp.maximum(
        jnp.dot(x, wct_ref[...], preferred_element_type=jnp.float32)
        + bc_ref[...], 0.0) * exists_ref[...]
    a_ref[...] = jnp.dot(cf, w1t_ref[...], preferred_element_type=jnp.float32)
    b_ref[...] = jnp.dot(cf, w2t_ref[...], preferred_element_type=jnp.float32)
    ps = jnp.sum(cf, axis=0, keepdims=True)
    es = jnp.sum(exists_ref[...], axis=0, keepdims=True)

    @pl.when(i == 0)
    def _():
        psum_ref[...] = ps
        esum_ref[...] = es

    @pl.when(i > 0)
    def _():
        psum_ref[...] += ps
        esum_ref[...] += es


def _tc_pre(child, exists, wct, bc, w1t, w2t):
    blk = 2000
    grid = (N // blk,)
    return pl.pallas_call(
        _tc_pre_body,
        grid=grid,
        in_specs=[
            pl.BlockSpec((blk, DF), lambda i: (i, 0)),
            pl.BlockSpec((blk, 1), lambda i: (i, 0)),
            pl.BlockSpec((DF, DH), lambda i: (0, 0)),
            pl.BlockSpec((1, DH), lambda i: (0, 0)),
            pl.BlockSpec((DH, DH), lambda i: (0, 0)),
            pl.BlockSpec((DH, DH), lambda i: (0, 0)),
        ],
        out_specs=[
            pl.BlockSpec((blk, DH), lambda i: (i, 0)),
            pl.BlockSpec((blk, DH), lambda i: (i, 0)),
            pl.BlockSpec((1, DH), lambda i: (0, 0)),
            pl.BlockSpec((1, 1), lambda i: (0, 0)),
        ],
        out_shape=[
            jax.ShapeDtypeStruct((N, DH), jnp.float32),
            jax.ShapeDtypeStruct((N, DH), jnp.float32),
            jax.ShapeDtypeStruct((1, DH), jnp.float32),
            jax.ShapeDtypeStruct((1, 1), jnp.float32),
        ],
    )(child, exists, wct, bc, w1t, w2t)


def _tc_edgeproj_body(ef_ref, w3t_ref, bne_ref, c_ref):
    c_ref[...] = (
        jnp.dot(ef_ref[...], w3t_ref[...], preferred_element_type=jnp.float32)
        + bne_ref[...])


def _tc_edgeproj(ef, w3t, bne):
    blk = 8000
    return pl.pallas_call(
        _tc_edgeproj_body,
        grid=(E // blk,),
        in_specs=[
            pl.BlockSpec((blk, DEF), lambda i: (i, 0)),
            pl.BlockSpec((DEF, DH), lambda i: (0, 0)),
            pl.BlockSpec((1, DH), lambda i: (0, 0)),
        ],
        out_specs=pl.BlockSpec((blk, DH), lambda i: (i, 0)),
        out_shape=jax.ShapeDtypeStruct((E, DH), jnp.float32),
    )(ef, w3t, bne)


def _tc_mid_body(p0_ref, p1_ref, cnt_ref, ones_ref, w1t_ref, w2t_ref,
                 a_ref, b_ref, psum_ref):
    i = pl.program_id(0)
    counts = lax.dot_general(cnt_ref[...], ones_ref[...],
                             (((0,), (0,)), ((), ())),
                             preferred_element_type=jnp.float32)
    cf = (p0_ref[...] + p1_ref[...]) / jnp.maximum(counts, 1.0)
    a_ref[...] = jnp.dot(cf, w1t_ref[...], preferred_element_type=jnp.float32)
    b_ref[...] = jnp.dot(cf, w2t_ref[...], preferred_element_type=jnp.float32)
    ps = jnp.sum(cf, axis=0, keepdims=True)

    @pl.when(i == 0)
    def _():
        psum_ref[...] = ps

    @pl.when(i > 0)
    def _():
        psum_ref[...] += ps


def _tc_mid(p0, p1, cnts, ones, w1t, w2t):
    blk = 2048
    return pl.pallas_call(
        _tc_mid_body,
        grid=(NPAD // blk,),
        in_specs=[
            pl.BlockSpec((blk, DH), lambda i: (i, 0)),
            pl.BlockSpec((blk, DH), lambda i: (i, 0)),
            pl.BlockSpec((NW, blk), lambda i: (0, i)),
            pl.BlockSpec((NW, 1), lambda i: (0, 0)),
            pl.BlockSpec((DH, DH), lambda i: (0, 0)),
            pl.BlockSpec((DH, DH), lambda i: (0, 0)),
        ],
        out_specs=[
            pl.BlockSpec((blk, DH), lambda i: (i, 0)),
            pl.BlockSpec((blk, DH), lambda i: (i, 0)),
            pl.BlockSpec((1, DH), lambda i: (0, 0)),
        ],
        out_shape=[
            jax.ShapeDtypeStruct((NPAD, DH), jnp.float32),
            jax.ShapeDtypeStruct((NPAD, DH), jnp.float32),
            jax.ShapeDtypeStruct((1, DH), jnp.float32),
        ],
    )(p0, p1, cnts, ones, w1t, w2t)


def _tc_head_body(p0_ref, p1_ref, p2_ref, esum_ref, wt0_ref, wt1_ref, wt2_ref,
                  bp_ref, out_ref):
    acc = (jnp.dot(p0_ref[...], wt0_ref[...], preferred_element_type=jnp.float32)
           + jnp.dot(p1_ref[...], wt1_ref[...], preferred_element_type=jnp.float32)
           + jnp.dot(p2_ref[...], wt2_ref[...], preferred_element_type=jnp.float32))
    out_ref[...] = jnp.maximum(acc / esum_ref[0, 0] + bp_ref[...], 0.0)


def _tc_head(p0, p1, p2, esum, wt0, wt1, wt2, bp):
    return pl.pallas_call(
        _tc_head_body,
        out_shape=jax.ShapeDtypeStruct((1, DF), jnp.float32),
    )(p0, p1, p2, esum, wt0, wt1, wt2, bp)


_sc_mesh = plsc.VectorSubcoreMesh(
    core_axis_name="c", subcore_axis_name="s", num_cores=NC, num_subcores=NS)


@functools.partial(
    pl.kernel,
    out_type=(jax.ShapeDtypeStruct((NC * NPAD, DH), jnp.float32),
              jax.ShapeDtypeStruct((NW * NPAD,), jnp.float32)),
    mesh=_sc_mesh,
    compiler_params=pltpu.CompilerParams(needs_layout_passes=False),
    scratch_types=[
        pltpu.VMEM((K,), jnp.int32),
        pltpu.VMEM((K,), jnp.int32),
        pltpu.VMEM((K, DH), jnp.float32),
        pltpu.VMEM((K, DH), jnp.float32),
        pltpu.VMEM((K, DH), jnp.float32),
        pltpu.VMEM((K, DH), jnp.float32),
        pltpu.VMEM((NPAD,), jnp.float32),
        pltpu.VMEM_SHARED((NPAD, DH), jnp.float32),
        pltpu.SemaphoreType.DMA,
        pltpu.SemaphoreType.DMA,
    ],
)
def _sc_edge(a_hbm, b_hbm, c_hbm, src_hbm, dst_hbm, out_hbm, outc_hbm,
             src_v, dst_v, rows_a, rows_b, rows_c, out_v, cnt_v, acc_sh,
             sem_a, sem_b):
    cid = lax.axis_index("c")
    sid = lax.axis_index("s")
    wid = cid * NS + sid

    zero16 = jnp.zeros((16,), jnp.float32)
    one16 = jnp.ones((16,), jnp.float32)
    tail_mask = lax.iota(jnp.int32, 16) >= 8

    def _zcnt(r, carry):
        cnt_v[pl.ds(r * 16, 16)] = zero16
        return carry

    lax.fori_loop(0, NPAD // 16, _zcnt, 0)

    def _zrow(r, carry):
        for c in range(DH // 16):
            out_v[r, pl.ds(c * 16, 16)] = zero16
        return carry

    lax.fori_loop(0, K, _zrow, 0)

    for z in range(RPT // K):
        pltpu.sync_copy(out_v, acc_sh.at[pl.ds(sid * RPT + z * K, K)])
    plsc.subcore_barrier()

    ebase = wid * EPW

    def _block(j, carry):
        eb = ebase + j * K
        pltpu.sync_copy(src_hbm.at[pl.ds(eb, K)], src_v)
        pltpu.sync_copy(dst_hbm.at[pl.ds(eb, K)], dst_v)
        cp_a = pltpu.async_copy(a_hbm.at[src_v], rows_a, sem_a)
        cp_b = pltpu.async_copy(b_hbm.at[dst_v], rows_b, sem_b)
        pltpu.sync_copy(c_hbm.at[pl.ds(eb, K)], rows_c)
        plsc.addupdate_scatter(cnt_v, [src_v[pl.ds(0, 16)]], one16)
        plsc.addupdate_scatter(cnt_v, [src_v[pl.ds(16, 16)]], one16)
        plsc.addupdate_scatter(cnt_v, [src_v[pl.ds(24, 16)]], one16,
                               mask=tail_mask)
        cp_a.wait()
        cp_b.wait()

        def _row(r, rc):
            for c in range(DH // 16):
                s = pl.ds(c * 16, 16)
                out_v[r, s] = jnp.maximum(
                    rows_a[r, s] + rows_b[r, s] + rows_c[r, s], 0.0)
            return rc

        lax.fori_loop(0, K, _row, 0)
        pltpu.sync_copy(out_v, acc_sh.at[src_v], add=True)
        return carry

    lax.fori_loop(0, NBLK, _block, 0)

    pltpu.sync_copy(cnt_v, outc_hbm.at[pl.ds(wid * NPAD, NPAD)])
    plsc.subcore_barrier()

    for z in range(RPT // K):
        rs = sid * RPT + z * K
        pltpu.sync_copy(acc_sh.at[pl.ds(rs, K)], rows_a)
        pltpu.sync_copy(rows_a, out_hbm.at[pl.ds(cid * NPAD + rs, K)])


def kernel(child_feats, child_exists, edge_type_onehot, edge_feats,
           edge_indices, W_child, b_child, W_ne, b_ne, W_parent, b_parent):
    child = child_feats[0]
    exists = child_exists[0]
    ef = jnp.concatenate([edge_type_onehot[0], edge_feats[0]], axis=1)
    src = edge_indices[0, :, 0]
    dst = edge_indices[0, :, 1]

    wct = W_child.T
    w1t = W_ne[:, 0:DH].T
    w2t = W_ne[:, DH:2 * DH].T
    w3t = W_ne[:, 2 * DH:2 * DH + DEF].T
    bc = b_child.reshape(1, DH)
    bne = b_ne.reshape(1, DH)
    wt0 = W_parent[:, 0:DH].T
    wt1 = W_parent[:, DH:2 * DH].T
    wt2 = W_parent[:, 2 * DH:3 * DH].T
    bp = b_parent.reshape(1, DF)

    a0, b0, psum0, esum = _tc_pre(child, exists, wct, bc, w1t, w2t)
    c = _tc_edgeproj(ef, w3t, bne)

    ones_nw = jnp.ones((NW, 1), jnp.float32)
    sums1, cnts1 = _sc_edge(a0, b0, c, src, dst)
    a1, b1, psum1 = _tc_mid(sums1[0:NPAD], sums1[NPAD:2 * NPAD],
                            cnts1.reshape(NW, NPAD), ones_nw, w1t, w2t)

    sums2, cnts2 = _sc_edge(a1, b1, c, src, dst)
    _, _, psum2 = _tc_mid(sums2[0:NPAD], sums2[NPAD:2 * NPAD],
                          cnts2.reshape(NW, NPAD), ones_nw, w1t, w2t)

    return _tc_head(psum0, psum1, psum2, esum, wt0, wt1, wt2, bp)


# speedup vs baseline: 4.3406x; 1.4195x over previous
import functools

import jax
import jax.numpy as jnp
from jax import lax
from jax.experimental import pallas as pl
from jax.experimental.pallas import tpu as pltpu
from jax.experimental.pallas import tpu_sc as plsc

N = 10000
E = 320000
DF = 128
DH = 128
DEF = 20
ACC_W = 144

NC, NS = 2, 16
NW = NC * NS
EPW = E // NW
K = 40
NBLK = EPW // K
KC = 80
NPAD = 10240
RPT = NPAD // NS
RZC = 128


def _tc_pre_body(child_ref, exists_ref, wct_ref, bc_ref, w1t_ref, w2t_ref,
                 a_ref, b_ref, psum_ref, esum_ref):
    i = pl.program_id(0)
    x = child_ref[...]
    cf = jnp.maximum(
        jnp.dot(x, wct_ref[...], preferred_element_type=jnp.float32)
        + bc_ref[...], 0.0) * exists_ref[...]
    a_ref[...] = jnp.dot(cf, w1t_ref[...], preferred_element_type=jnp.float32)
    b_ref[...] = jnp.dot(cf, w2t_ref[...], preferred_element_type=jnp.float32)
    ps = jnp.sum(cf, axis=0, keepdims=True)
    es = jnp.sum(exists_ref[...], axis=0, keepdims=True)

    @pl.when(i == 0)
    def _():
        psum_ref[...] = ps
        esum_ref[...] = es

    @pl.when(i > 0)
    def _():
        psum_ref[...] += ps
        esum_ref[...] += es


def _tc_pre(child, exists, wct, bc, w1t, w2t):
    blk = 2000
    grid = (N // blk,)
    return pl.pallas_call(
        _tc_pre_body,
        grid=grid,
        in_specs=[
            pl.BlockSpec((blk, DF), lambda i: (i, 0)),
            pl.BlockSpec((blk, 1), lambda i: (i, 0)),
            pl.BlockSpec((DF, DH), lambda i: (0, 0)),
            pl.BlockSpec((1, DH), lambda i: (0, 0)),
            pl.BlockSpec((DH, DH), lambda i: (0, 0)),
            pl.BlockSpec((DH, DH), lambda i: (0, 0)),
        ],
        out_specs=[
            pl.BlockSpec((blk, DH), lambda i: (i, 0)),
            pl.BlockSpec((blk, DH), lambda i: (i, 0)),
            pl.BlockSpec((1, DH), lambda i: (0, 0)),
            pl.BlockSpec((1, 1), lambda i: (0, 0)),
        ],
        out_shape=[
            jax.ShapeDtypeStruct((N, DH), jnp.float32),
            jax.ShapeDtypeStruct((N, DH), jnp.float32),
            jax.ShapeDtypeStruct((1, DH), jnp.float32),
            jax.ShapeDtypeStruct((1, 1), jnp.float32),
        ],
    )(child, exists, wct, bc, w1t, w2t)


def _tc_edgeproj_body(ef_ref, w3t_ref, bne_ref, c_ref):
    c_ref[...] = (
        jnp.dot(ef_ref[...], w3t_ref[...], preferred_element_type=jnp.float32)
        + bne_ref[...])


def _tc_edgeproj(ef, w3t, bne):
    blk = 8000
    return pl.pallas_call(
        _tc_edgeproj_body,
        grid=(E // blk,),
        in_specs=[
            pl.BlockSpec((blk, DEF), lambda i: (i, 0)),
            pl.BlockSpec((DEF, DH), lambda i: (0, 0)),
            pl.BlockSpec((1, DH), lambda i: (0, 0)),
        ],
        out_specs=pl.BlockSpec((blk, DH), lambda i: (i, 0)),
        out_shape=jax.ShapeDtypeStruct((E, DH), jnp.float32),
    )(ef, w3t, bne)


def _tc_mid_body(p0_ref, p1_ref, cnt_ref, ones_ref, w1t_ref, w2t_ref,
                 a_ref, b_ref, psum_ref):
    i = pl.program_id(0)
    counts = lax.dot_general(cnt_ref[...], ones_ref[...],
                             (((0,), (0,)), ((), ())),
                             preferred_element_type=jnp.float32)
    cf = (p0_ref[...] + p1_ref[...]) / jnp.maximum(counts, 1.0)
    a_ref[...] = jnp.dot(cf, w1t_ref[...], preferred_element_type=jnp.float32)
    b_ref[...] = jnp.dot(cf, w2t_ref[...], preferred_element_type=jnp.float32)
    ps = jnp.sum(cf, axis=0, keepdims=True)

    @pl.when(i == 0)
    def _():
        psum_ref[...] = ps

    @pl.when(i > 0)
    def _():
        psum_ref[...] += ps


def _tc_mid(p0, p1, cnts, ones, w1t, w2t):
    blk = 2048
    return pl.pallas_call(
        _tc_mid_body,
        grid=(NPAD // blk,),
        in_specs=[
            pl.BlockSpec((blk, DH), lambda i: (i, 0)),
            pl.BlockSpec((blk, DH), lambda i: (i, 0)),
            pl.BlockSpec((NW, blk), lambda i: (0, i)),
            pl.BlockSpec((NW, 1), lambda i: (0, 0)),
            pl.BlockSpec((DH, DH), lambda i: (0, 0)),
            pl.BlockSpec((DH, DH), lambda i: (0, 0)),
        ],
        out_specs=[
            pl.BlockSpec((blk, DH), lambda i: (i, 0)),
            pl.BlockSpec((blk, DH), lambda i: (i, 0)),
            pl.BlockSpec((1, DH), lambda i: (0, 0)),
        ],
        out_shape=[
            jax.ShapeDtypeStruct((NPAD, DH), jnp.float32),
            jax.ShapeDtypeStruct((NPAD, DH), jnp.float32),
            jax.ShapeDtypeStruct((1, DH), jnp.float32),
        ],
    )(p0, p1, cnts, ones, w1t, w2t)


def _tc_head_body(p0_ref, p1_ref, p2_ref, esum_ref, wt0_ref, wt1_ref, wt2_ref,
                  bp_ref, out_ref):
    acc = (jnp.dot(p0_ref[...], wt0_ref[...], preferred_element_type=jnp.float32)
           + jnp.dot(p1_ref[...], wt1_ref[...], preferred_element_type=jnp.float32)
           + jnp.dot(p2_ref[...], wt2_ref[...], preferred_element_type=jnp.float32))
    out_ref[...] = jnp.maximum(acc / esum_ref[0, 0] + bp_ref[...], 0.0)


def _tc_head(p0, p1, p2, esum, wt0, wt1, wt2, bp):
    return pl.pallas_call(
        _tc_head_body,
        out_shape=jax.ShapeDtypeStruct((1, DF), jnp.float32),
    )(p0, p1, p2, esum, wt0, wt1, wt2, bp)


_sc_mesh = plsc.VectorSubcoreMesh(
    core_axis_name="c", subcore_axis_name="s", num_cores=NC, num_subcores=NS)


@functools.partial(
    pl.kernel,
    out_type=jax.ShapeDtypeStruct((NW * NPAD,), jnp.float32),
    mesh=_sc_mesh,
    compiler_params=pltpu.CompilerParams(needs_layout_passes=False),
    scratch_types=[
        pltpu.VMEM((KC,), jnp.int32),
        pltpu.VMEM((NPAD,), jnp.float32),
    ],
)
def _sc_count(src_hbm, outc_hbm, src_v, cnt_v):
    cid = lax.axis_index("c")
    sid = lax.axis_index("s")
    wid = cid * NS + sid

    zero16 = jnp.zeros((16,), jnp.float32)
    one16 = jnp.ones((16,), jnp.float32)

    def _zcnt(r, carry):
        cnt_v[pl.ds(r * 16, 16)] = zero16
        return carry

    lax.fori_loop(0, NPAD // 16, _zcnt, 0)

    ebase = wid * EPW

    def _block(j, carry):
        pltpu.sync_copy(src_hbm.at[pl.ds(ebase + j * KC, KC)], src_v)
        for q in range(KC // 16):
            plsc.addupdate_scatter(cnt_v, [src_v[pl.ds(q * 16, 16)]], one16)
        return carry

    lax.fori_loop(0, EPW // KC, _block, 0)
    pltpu.sync_copy(cnt_v, outc_hbm.at[pl.ds(wid * NPAD, NPAD)])


@functools.partial(
    pl.kernel,
    out_type=jax.ShapeDtypeStruct((NC * NPAD, DH), jnp.float32),
    mesh=_sc_mesh,
    compiler_params=pltpu.CompilerParams(needs_layout_passes=False),
    scratch_types=[
        pltpu.VMEM((K,), jnp.int32),
        pltpu.VMEM((K,), jnp.int32),
        pltpu.VMEM((K, DH), jnp.float32),
        pltpu.VMEM((K, DH), jnp.float32),
        pltpu.VMEM((K, DH), jnp.float32),
        pltpu.VMEM((K,), jnp.int32),
        pltpu.VMEM((K,), jnp.int32),
        pltpu.VMEM((K, DH), jnp.float32),
        pltpu.VMEM((K, DH), jnp.float32),
        pltpu.VMEM((K, DH), jnp.float32),
        pltpu.VMEM((K, DH), jnp.float32),
        pltpu.VMEM_SHARED((NPAD, DH), jnp.float32),
        pltpu.SemaphoreType.DMA,
        pltpu.SemaphoreType.DMA,
        pltpu.SemaphoreType.DMA,
        pltpu.SemaphoreType.DMA,
        pltpu.SemaphoreType.DMA,
        pltpu.SemaphoreType.DMA,
    ],
)
def _sc_edge(a_hbm, b_hbm, c_hbm, src_hbm, dst_hbm, out_hbm,
             src0, dst0, ra0, rb0, rc0, src1, dst1, ra1, rb1, rc1,
             out_v, acc_sh, sa0, sb0, sc0, sa1, sb1, sc1):
    cid = lax.axis_index("c")
    sid = lax.axis_index("s")
    wid = cid * NS + sid

    srcs = (src0, src1)
    dsts = (dst0, dst1)
    ras = (ra0, ra1)
    rbs = (rb0, rb1)
    rcs = (rc0, rc1)
    sas = (sa0, sa1)
    sbs = (sb0, sb1)
    scs = (sc0, sc1)

    zero16 = jnp.zeros((16,), jnp.float32)

    def _zrow(r, carry):
        for c in range(DH // 16):
            out_v[r, pl.ds(c * 16, 16)] = zero16
        return carry

    lax.fori_loop(0, K, _zrow, 0)
    for z in range(RPT // K):
        pltpu.sync_copy(out_v, acc_sh.at[pl.ds(sid * RPT + z * K, K)])
    plsc.subcore_barrier()

    ebase = wid * EPW

    def _issue(j, b):
        eb = ebase + j * K
        pltpu.sync_copy(src_hbm.at[pl.ds(eb, K)], srcs[b])
        pltpu.sync_copy(dst_hbm.at[pl.ds(eb, K)], dsts[b])
        pltpu.async_copy(a_hbm.at[srcs[b]], ras[b], sas[b])
        pltpu.async_copy(b_hbm.at[dsts[b]], rbs[b], sbs[b])
        pltpu.async_copy(c_hbm.at[pl.ds(eb, K)], rcs[b], scs[b])

    _issue(0, 0)
    _issue(1, 1)

    def _round(g, carry):
        for b in range(2):
            j = 2 * g + b
            pltpu.make_async_copy(a_hbm.at[srcs[b]], ras[b], sas[b]).wait()
            pltpu.make_async_copy(b_hbm.at[dsts[b]], rbs[b], sbs[b]).wait()
            pltpu.make_async_copy(
                c_hbm.at[pl.ds(0, K)], rcs[b], scs[b]).wait()
            ra, rb, rc = ras[b], rbs[b], rcs[b]

            def _row(r, rc_):
                for c in range(DH // 16):
                    s = pl.ds(c * 16, 16)
                    out_v[r, s] = jnp.maximum(
                        ra[r, s] + rb[r, s] + rc[r, s], 0.0)
                return rc_

            lax.fori_loop(0, K, _row, 0)
            pltpu.sync_copy(out_v, acc_sh.at[srcs[b]], add=True)

            @pl.when(j + 2 < NBLK)
            def _():
                _issue(j + 2, b)
        return carry

    lax.fori_loop(0, NBLK // 2, _round, 0)
    plsc.subcore_barrier()

    for z in range(RPT // K):
        rs = sid * RPT + z * K
        pltpu.sync_copy(acc_sh.at[pl.ds(rs, K)], ra0)
        pltpu.sync_copy(ra0, out_hbm.at[pl.ds(cid * NPAD + rs, K)])


def kernel(child_feats, child_exists, edge_type_onehot, edge_feats,
           edge_indices, W_child, b_child, W_ne, b_ne, W_parent, b_parent):
    child = child_feats[0]
    exists = child_exists[0]
    ef = jnp.concatenate([edge_type_onehot[0], edge_feats[0]], axis=1)
    src = edge_indices[0, :, 0]
    dst = edge_indices[0, :, 1]

    wct = W_child.T
    w1t = W_ne[:, 0:DH].T
    w2t = W_ne[:, DH:2 * DH].T
    w3t = W_ne[:, 2 * DH:2 * DH + DEF].T
    bc = b_child.reshape(1, DH)
    bne = b_ne.reshape(1, DH)
    wt0 = W_parent[:, 0:DH].T
    wt1 = W_parent[:, DH:2 * DH].T
    wt2 = W_parent[:, 2 * DH:3 * DH].T
    bp = b_parent.reshape(1, DF)

    a0, b0, psum0, esum = _tc_pre(child, exists, wct, bc, w1t, w2t)
    c = _tc_edgeproj(ef, w3t, bne)

    ones_nw = jnp.ones((NW, 1), jnp.float32)
    cnts = _sc_count(src).reshape(NW, NPAD)

    sums1 = _sc_edge(a0, b0, c, src, dst)
    a1, b1, psum1 = _tc_mid(sums1[0:NPAD], sums1[NPAD:2 * NPAD],
                            cnts, ones_nw, w1t, w2t)

    sums2 = _sc_edge(a1, b1, c, src, dst)
    _, _, psum2 = _tc_mid(sums2[0:NPAD], sums2[NPAD:2 * NPAD],
                          cnts, ones_nw, w1t, w2t)

    return _tc_head(psum0, psum1, psum2, esum, wt0, wt1, wt2, bp)


# speedup vs baseline: 5.7787x; 1.3313x over previous
import functools

import jax
import jax.numpy as jnp
from jax import lax
from jax.experimental import pallas as pl
from jax.experimental.pallas import tpu as pltpu
from jax.experimental.pallas import tpu_sc as plsc

N = 10000
E = 320000
DF = 128
DH = 128
DEF = 20
ACC_W = 144

NC, NS = 2, 16
NW = NC * NS
EPW = E // NW
K = 40
NBLK = EPW // K
KC = 80
NPAD = 10240
RPT = NPAD // NS
RZC = 128


def _tc_pre_body(child_ref, exists_ref, wct_ref, bc_ref, w1t_ref, w2t_ref,
                 a_ref, b_ref, psum_ref, esum_ref):
    i = pl.program_id(0)
    x = child_ref[...]
    cf = jnp.maximum(
        jnp.dot(x, wct_ref[...], preferred_element_type=jnp.float32)
        + bc_ref[...], 0.0) * exists_ref[...]
    a_ref[...] = jnp.dot(cf, w1t_ref[...], preferred_element_type=jnp.float32)
    b_ref[...] = jnp.dot(cf, w2t_ref[...], preferred_element_type=jnp.float32)
    ps = jnp.sum(cf, axis=0, keepdims=True)
    es = jnp.sum(exists_ref[...], axis=0, keepdims=True)

    @pl.when(i == 0)
    def _():
        psum_ref[...] = ps
        esum_ref[...] = es

    @pl.when(i > 0)
    def _():
        psum_ref[...] += ps
        esum_ref[...] += es


def _tc_pre(child, exists, wct, bc, w1t, w2t):
    blk = 2000
    grid = (N // blk,)
    return pl.pallas_call(
        _tc_pre_body,
        grid=grid,
        in_specs=[
            pl.BlockSpec((blk, DF), lambda i: (i, 0)),
            pl.BlockSpec((blk, 1), lambda i: (i, 0)),
            pl.BlockSpec((DF, DH), lambda i: (0, 0)),
            pl.BlockSpec((1, DH), lambda i: (0, 0)),
            pl.BlockSpec((DH, DH), lambda i: (0, 0)),
            pl.BlockSpec((DH, DH), lambda i: (0, 0)),
        ],
        out_specs=[
            pl.BlockSpec((blk, DH), lambda i: (i, 0)),
            pl.BlockSpec((blk, DH), lambda i: (i, 0)),
            pl.BlockSpec((1, DH), lambda i: (0, 0)),
            pl.BlockSpec((1, 1), lambda i: (0, 0)),
        ],
        out_shape=[
            jax.ShapeDtypeStruct((N, DH), jnp.float32),
            jax.ShapeDtypeStruct((N, DH), jnp.float32),
            jax.ShapeDtypeStruct((1, DH), jnp.float32),
            jax.ShapeDtypeStruct((1, 1), jnp.float32),
        ],
    )(child, exists, wct, bc, w1t, w2t)


def _tc_edgeproj_body(ef_ref, w3t_ref, bne_ref, c_ref):
    c_ref[...] = (
        jnp.dot(ef_ref[...], w3t_ref[...], preferred_element_type=jnp.float32)
        + bne_ref[...])


def _tc_edgeproj(ef, w3t, bne):
    blk = 8000
    return pl.pallas_call(
        _tc_edgeproj_body,
        grid=(E // blk,),
        in_specs=[
            pl.BlockSpec((blk, DEF), lambda i: (i, 0)),
            pl.BlockSpec((DEF, DH), lambda i: (0, 0)),
            pl.BlockSpec((1, DH), lambda i: (0, 0)),
        ],
        out_specs=pl.BlockSpec((blk, DH), lambda i: (i, 0)),
        out_shape=jax.ShapeDtypeStruct((E, DH), jnp.float32),
    )(ef, w3t, bne)


def _tc_mid_body(p0_ref, p1_ref, cnt_ref, ones_ref, w1t_ref, w2t_ref,
                 a_ref, b_ref, psum_ref):
    i = pl.program_id(0)
    counts = lax.dot_general(cnt_ref[...], ones_ref[...],
                             (((0,), (0,)), ((), ())),
                             preferred_element_type=jnp.float32)
    cf = (p0_ref[...] + p1_ref[...]) / jnp.maximum(counts, 1.0)
    a_ref[...] = jnp.dot(cf, w1t_ref[...], preferred_element_type=jnp.float32)
    b_ref[...] = jnp.dot(cf, w2t_ref[...], preferred_element_type=jnp.float32)
    ps = jnp.sum(cf, axis=0, keepdims=True)

    @pl.when(i == 0)
    def _():
        psum_ref[...] = ps

    @pl.when(i > 0)
    def _():
        psum_ref[...] += ps


def _tc_mid(p0, p1, cnts, ones, w1t, w2t):
    blk = 2048
    return pl.pallas_call(
        _tc_mid_body,
        grid=(NPAD // blk,),
        in_specs=[
            pl.BlockSpec((blk, DH), lambda i: (i, 0)),
            pl.BlockSpec((blk, DH), lambda i: (i, 0)),
            pl.BlockSpec((NW, blk), lambda i: (0, i)),
            pl.BlockSpec((NW, 1), lambda i: (0, 0)),
            pl.BlockSpec((DH, DH), lambda i: (0, 0)),
            pl.BlockSpec((DH, DH), lambda i: (0, 0)),
        ],
        out_specs=[
            pl.BlockSpec((blk, DH), lambda i: (i, 0)),
            pl.BlockSpec((blk, DH), lambda i: (i, 0)),
            pl.BlockSpec((1, DH), lambda i: (0, 0)),
        ],
        out_shape=[
            jax.ShapeDtypeStruct((NPAD, DH), jnp.float32),
            jax.ShapeDtypeStruct((NPAD, DH), jnp.float32),
            jax.ShapeDtypeStruct((1, DH), jnp.float32),
        ],
    )(p0, p1, cnts, ones, w1t, w2t)


def _tc_head_body(p0_ref, p1_ref, p2_ref, esum_ref, wt0_ref, wt1_ref, wt2_ref,
                  bp_ref, out_ref):
    acc = (jnp.dot(p0_ref[...], wt0_ref[...], preferred_element_type=jnp.float32)
           + jnp.dot(p1_ref[...], wt1_ref[...], preferred_element_type=jnp.float32)
           + jnp.dot(p2_ref[...], wt2_ref[...], preferred_element_type=jnp.float32))
    out_ref[...] = jnp.maximum(acc / esum_ref[0, 0] + bp_ref[...], 0.0)


def _tc_head(p0, p1, p2, esum, wt0, wt1, wt2, bp):
    return pl.pallas_call(
        _tc_head_body,
        out_shape=jax.ShapeDtypeStruct((1, DF), jnp.float32),
    )(p0, p1, p2, esum, wt0, wt1, wt2, bp)


_sc_mesh = plsc.VectorSubcoreMesh(
    core_axis_name="c", subcore_axis_name="s", num_cores=NC, num_subcores=NS)


@functools.partial(
    pl.kernel,
    out_type=jax.ShapeDtypeStruct((NW * NPAD,), jnp.float32),
    mesh=_sc_mesh,
    compiler_params=pltpu.CompilerParams(needs_layout_passes=False),
    scratch_types=[
        pltpu.VMEM((KC,), jnp.int32),
        pltpu.VMEM((NPAD,), jnp.float32),
    ],
)
def _sc_count(src_hbm, outc_hbm, src_v, cnt_v):
    cid = lax.axis_index("c")
    sid = lax.axis_index("s")
    wid = cid * NS + sid

    zero16 = jnp.zeros((16,), jnp.float32)
    one16 = jnp.ones((16,), jnp.float32)

    def _zcnt(r, carry):
        cnt_v[pl.ds(r * 16, 16)] = zero16
        return carry

    lax.fori_loop(0, NPAD // 16, _zcnt, 0)

    ebase = wid * EPW

    def _block(j, carry):
        pltpu.sync_copy(src_hbm.at[pl.ds(ebase + j * KC, KC)], src_v)
        for q in range(KC // 16):
            plsc.addupdate_scatter(cnt_v, [src_v[pl.ds(q * 16, 16)]], one16)
        return carry

    lax.fori_loop(0, EPW // KC, _block, 0)
    pltpu.sync_copy(cnt_v, outc_hbm.at[pl.ds(wid * NPAD, NPAD)])


@functools.partial(
    pl.kernel,
    out_type=jax.ShapeDtypeStruct((NC * NPAD, DH), jnp.float32),
    mesh=_sc_mesh,
    compiler_params=pltpu.CompilerParams(needs_layout_passes=False),
    scratch_types=[
        pltpu.VMEM((K,), jnp.int32),
        pltpu.VMEM((K,), jnp.int32),
        pltpu.VMEM((K, DH), jnp.float32),
        pltpu.VMEM((K, DH), jnp.float32),
        pltpu.VMEM((K, DH), jnp.float32),
        pltpu.VMEM((K,), jnp.int32),
        pltpu.VMEM((K,), jnp.int32),
        pltpu.VMEM((K, DH), jnp.float32),
        pltpu.VMEM((K, DH), jnp.float32),
        pltpu.VMEM((K, DH), jnp.float32),
        pltpu.VMEM((K, DH), jnp.float32),
        pltpu.VMEM((K,), jnp.int32),
        pltpu.VMEM((K,), jnp.int32),
        pltpu.VMEM_SHARED((NPAD, DH), jnp.float32),
        pltpu.SemaphoreType.DMA,
        pltpu.SemaphoreType.DMA,
        pltpu.SemaphoreType.DMA,
        pltpu.SemaphoreType.DMA,
        pltpu.SemaphoreType.DMA,
        pltpu.SemaphoreType.DMA,
        pltpu.SemaphoreType.DMA,
        pltpu.SemaphoreType.DMA,
        pltpu.SemaphoreType.DMA,
        pltpu.SemaphoreType.DMA,
    ],
)
def _sc_edge(a_hbm, b_hbm, c_hbm, src_hbm, dst_hbm, out_hbm,
             src0, dst0, ra0, rb0, rc0, src1, dst1, ra1, rb1, rc1,
             out_v, sx0, sx1, acc_sh,
             sa0, sb0, sc0, sa1, sb1, sc1, si0, si1, sd0, sd1):
    cid = lax.axis_index("c")
    sid = lax.axis_index("s")
    wid = cid * NS + sid

    srcs = (src0, src1)
    dsts = (dst0, dst1)
    ras = (ra0, ra1)
    rbs = (rb0, rb1)
    rcs = (rc0, rc1)
    sxs = (sx0, sx1)
    sas = (sa0, sa1)
    sbs = (sb0, sb1)
    scs = (sc0, sc1)
    sis = (si0, si1)
    sds = (sd0, sd1)

    zero16 = jnp.zeros((16,), jnp.float32)

    def _zrow(r, carry):
        for c in range(DH // 16):
            out_v[r, pl.ds(c * 16, 16)] = zero16
        return carry

    lax.fori_loop(0, K, _zrow, 0)
    for z in range(RPT // K):
        pltpu.sync_copy(out_v, acc_sh.at[pl.ds(sid * RPT + z * K, K)])
    plsc.subcore_barrier()

    ebase = wid * EPW

    def _issue_idx(j, b):
        eb = ebase + j * K
        pltpu.async_copy(src_hbm.at[pl.ds(eb, K)], srcs[b], sis[b])
        pltpu.async_copy(dst_hbm.at[pl.ds(eb, K)], dsts[b], sds[b])

    def _wait_idx(b):
        pltpu.make_async_copy(src_hbm.at[pl.ds(0, K)], srcs[b], sis[b]).wait()
        pltpu.make_async_copy(dst_hbm.at[pl.ds(0, K)], dsts[b], sds[b]).wait()

    def _issue_rows(j, b):
        eb = ebase + j * K
        pltpu.async_copy(a_hbm.at[srcs[b]], ras[b], sas[b])
        pltpu.async_copy(b_hbm.at[dsts[b]], rbs[b], sbs[b])
        pltpu.async_copy(c_hbm.at[pl.ds(eb, K)], rcs[b], scs[b])

    def _copy_sidx(b):
        sxs[b][pl.ds(0, 16)] = srcs[b][pl.ds(0, 16)]
        sxs[b][pl.ds(16, 16)] = srcs[b][pl.ds(16, 16)]
        sxs[b][pl.ds(24, 16)] = srcs[b][pl.ds(24, 16)]

    for b in range(2):
        _issue_idx(b, b)
        _wait_idx(b)
        _copy_sidx(b)
        _issue_rows(b, b)

    def _round(g, carry):
        for b in range(2):
            j = 2 * g + b
            pltpu.make_async_copy(a_hbm.at[srcs[b]], ras[b], sas[b]).wait()
            pltpu.make_async_copy(b_hbm.at[dsts[b]], rbs[b], sbs[b]).wait()
            pltpu.make_async_copy(c_hbm.at[pl.ds(0, K)], rcs[b],
                                  scs[b]).wait()

            @pl.when(j + 2 < NBLK)
            def _():
                _issue_idx(j + 2, b)

            ra, rb, rc = ras[b], rbs[b], rcs[b]

            def _row(r, rc_):
                for c in range(DH // 16):
                    s = pl.ds(c * 16, 16)
                    out_v[r, s] = jnp.maximum(
                        ra[r, s] + rb[r, s] + rc[r, s], 0.0)
                return rc_

            lax.fori_loop(0, K, _row, 0)
            pltpu.sync_copy(out_v, acc_sh.at[sxs[b]], add=True)

            @pl.when(j + 2 < NBLK)
            def _():
                _wait_idx(b)
                _copy_sidx(b)
                _issue_rows(j + 2, b)
        return carry

    lax.fori_loop(0, NBLK // 2, _round, 0)
    plsc.subcore_barrier()

    for z in range(RPT // K):
        rs = sid * RPT + z * K
        pltpu.sync_copy(acc_sh.at[pl.ds(rs, K)], out_v)
        pltpu.sync_copy(out_v, out_hbm.at[pl.ds(cid * NPAD + rs, K)])


def kernel(child_feats, child_exists, edge_type_onehot, edge_feats,
           edge_indices, W_child, b_child, W_ne, b_ne, W_parent, b_parent):
    child = child_feats[0]
    exists = child_exists[0]
    ef = jnp.concatenate([edge_type_onehot[0], edge_feats[0]], axis=1)
    src = edge_indices[0, :, 0]
    dst = edge_indices[0, :, 1]

    wct = W_child.T
    w1t = W_ne[:, 0:DH].T
    w2t = W_ne[:, DH:2 * DH].T
    w3t = W_ne[:, 2 * DH:2 * DH + DEF].T
    bc = b_child.reshape(1, DH)
    bne = b_ne.reshape(1, DH)
    wt0 = W_parent[:, 0:DH].T
    wt1 = W_parent[:, DH:2 * DH].T
    wt2 = W_parent[:, 2 * DH:3 * DH].T
    bp = b_parent.reshape(1, DF)

    a0, b0, psum0, esum = _tc_pre(child, exists, wct, bc, w1t, w2t)
    c = _tc_edgeproj(ef, w3t, bne)

    ones_nw = jnp.ones((NW, 1), jnp.float32)
    cnts = _sc_count(src).reshape(NW, NPAD)

    sums1 = _sc_edge(a0, b0, c, src, dst)
    a1, b1, psum1 = _tc_mid(sums1[0:NPAD], sums1[NPAD:2 * NPAD],
                            cnts, ones_nw, w1t, w2t)

    sums2 = _sc_edge(a1, b1, c, src, dst)
    _, _, psum2 = _tc_mid(sums2[0:NPAD], sums2[NPAD:2 * NPAD],
                          cnts, ones_nw, w1t, w2t)

    return _tc_head(psum0, psum1, psum2, esum, wt0, wt1, wt2, bp)
